# baseline (speedup 1.0000x reference)
import functools

import jax
import jax.numpy as jnp
from jax import lax
from jax.experimental import pallas as pl
from jax.experimental.pallas import tpu as pltpu
from jax.experimental.pallas import tpu_sc as plsc

_TILE = 256


def _rowsq_like_xla(x):
    t = x * x
    acc = t[:, 0:1]
    for j in range(1, t.shape[1]):
        acc = acc + t[:, j:j + 1]
    return acc


def _dist_argmin_body(x_ref, wt_ref, idx_ref):
    x = x_ref[...]
    wt = wt_ref[...]
    k = wt.shape[1]
    x_sq = _rowsq_like_xla(x)
    w_sq = jnp.sum(wt * wt, axis=0, keepdims=True)
    f = jnp.dot(x.astype(jnp.bfloat16), wt.astype(jnp.bfloat16),
                preferred_element_type=jnp.float32)
    d2 = x_sq - 2.0 * f + w_sq
    dist = jnp.sqrt(jnp.maximum(d2, 0.0))
    m = jnp.min(dist, axis=1, keepdims=True)
    cols = lax.broadcasted_iota(jnp.int32, dist.shape, 1)
    cand = jnp.where(dist == m, cols, k)
    idx_ref[0, 0, :] = jnp.min(cand, axis=1)


def _dist_argmin(flat, wt):
    n, c = flat.shape
    k = wt.shape[1]
    grid = n // _TILE
    out = pl.pallas_call(
        _dist_argmin_body,
        grid=(grid,),
        in_specs=[
            pl.BlockSpec((_TILE, c), lambda i: (i, 0)),
            pl.BlockSpec((c, k), lambda i: (0, 0)),
        ],
        out_specs=pl.BlockSpec((1, 1, _TILE), lambda i: (i, 0, 0)),
        out_shape=jax.ShapeDtypeStruct((grid, 1, _TILE), jnp.int32),
    )(flat, wt)
    return out.reshape(n)


def _make_sc_gather(n, c, k_codes):
    info = plsc.get_sparse_core_info()
    nw = info.num_cores * info.num_subcores
    b_w = n // nw
    n_chunks = b_w // 128
    rows = b_w * c // 128
    per_row = 128 // c
    mesh = plsc.VectorSubcoreMesh(core_axis_name="c", subcore_axis_name="s")

    @functools.partial(
        pl.kernel,
        mesh=mesh,
        out_type=[
            jax.ShapeDtypeStruct((nw, rows, 128), jnp.float32),
            jax.ShapeDtypeStruct((nw, rows, 128), jnp.float32),
        ],
        scratch_types=[
            pltpu.VMEM((n_chunks, 128), jnp.int32),
            pltpu.VMEM((b_w, 128), jnp.float32),
            pltpu.VMEM((rows, 128), jnp.float32),
            pltpu.VMEM((rows, 128), jnp.float32),
            pltpu.VMEM((rows, 128), jnp.float32),
            pltpu.SemaphoreType.DMA,
        ],
    )
    def sc_gather(wpad_hbm, idx_hbm, z_hbm, zq_hbm, st_hbm,
                  idx_v, g_v, z_v, zq_v, st_v, sem):
        wid = lax.axis_index("s") * info.num_cores + lax.axis_index("c")
        pltpu.sync_copy(idx_hbm.at[wid], idx_v)
        pltpu.sync_copy(z_hbm.at[wid], z_v)
        for j in range(n_chunks):
            pltpu.async_copy(
                wpad_hbm.at[idx_v.at[j]], g_v.at[pl.ds(j * 128, 128)], sem
            ).wait()

        def body(r, carry):
            for q in range(per_row):
                for ch in range(c // 16):
                    lane = q * c + ch * 16
                    zr = z_v[r, pl.ds(lane, 16)]
                    wr = g_v[r * per_row + q, pl.ds(ch * 16, 16)]
                    zq_v[r, pl.ds(lane, 16)] = wr
                    st_v[r, pl.ds(lane, 16)] = zr + (wr - zr)
            return carry

        lax.fori_loop(0, rows, body, 0)
        pltpu.sync_copy(zq_v, zq_hbm.at[wid])
        pltpu.sync_copy(st_v, st_hbm.at[wid])

    return sc_gather, nw, n_chunks, rows


def kernel(z_e, W):
    b, ch, hh, ww = z_e.shape
    k_codes, d_code = W.shape
    z = jnp.transpose(z_e, (0, 2, 3, 1))
    flat = z.reshape(-1, d_code)
    n = flat.shape[0]

    indices = _dist_argmin(flat, W.T)

    sc_gather, nw, n_chunks, rows = _make_sc_gather(n, d_code, k_codes)
    idx3 = indices.reshape(nw, n_chunks, 128)
    w_pad = jnp.pad(W, ((0, 0), (0, 128 - d_code)))
    z3 = flat.reshape(nw, rows, 128)
    z_q_c, z_q_st_c = sc_gather(w_pad, idx3, z3)

    z_q = z_q_c.reshape(b, hh, ww, ch)
    z_q_st = z_q_st_c.reshape(b, hh, ww, ch)
    z_q_out = jnp.transpose(z_q, (0, 3, 1, 2))
    z_q_st_out = jnp.transpose(z_q_st, (0, 3, 1, 2))
    return (z_q_st_out, z_q_out, indices.reshape(b, hh * ww))
